# baseline (speedup 1.0000x reference)
import jax
import jax.numpy as jnp
from jax import lax
from jax.experimental import pallas as pl
from jax.experimental.pallas import tpu as pltpu
from jax.experimental.pallas import tpu_sc as plsc

BLOCK = 8192
EMBED = 1024

_info = plsc.get_sparse_core_info()
_NC, _NS = _info.num_cores, _info.num_subcores
_NR = _NS // 2
_ROWS_PER_SC = BLOCK // _NC
_ROWS_PER_RD = _ROWS_PER_SC // _NR
_CHUNK = 64
_ROUNDS = _ROWS_PER_RD // _CHUNK


def _copy_body(pe_hbm, out_hbm, spmem, sem_in, sem_out):
    c = lax.axis_index("c")
    sid = lax.axis_index("s")
    slot = lax.rem(sid, _NR)
    base = c * _ROWS_PER_SC + slot * _ROWS_PER_RD
    is_reader = sid < _NR

    def in_copy(i):
        return pltpu.make_async_copy(
            pe_hbm.at[pl.ds(base + i * _CHUNK, _CHUNK)],
            spmem.at[i % 2, slot], sem_in)

    def out_copy(i):
        return pltpu.make_async_copy(
            spmem.at[i % 2, slot],
            out_hbm.at[pl.ds(base + i * _CHUNK, _CHUNK)], sem_out)

    for i in range(_ROUNDS):
        @pl.when(is_reader)
        def _read():
            cp = in_copy(i)
            cp.start()
            cp.wait()

        plsc.subcore_barrier()

        @pl.when(jnp.logical_not(is_reader))
        def _write():
            out_copy(i).start()
            if i > 0:
                out_copy(i - 1).wait()

    @pl.when(jnp.logical_not(is_reader))
    def _drain():
        out_copy(_ROUNDS - 1).wait()


def _sc_copy(pe):
    mesh = plsc.VectorSubcoreMesh(core_axis_name="c", subcore_axis_name="s")
    return pl.kernel(
        _copy_body,
        mesh=mesh,
        out_type=jax.ShapeDtypeStruct((BLOCK, EMBED), jnp.float32),
        scratch_types=[
            pltpu.VMEM_SHARED((2, _NR, _CHUNK, EMBED), jnp.float32),
            pltpu.SemaphoreType.DMA,
            pltpu.SemaphoreType.DMA,
        ],
    )(pe)


def kernel(x, pe):
    return _sc_copy(pe)


# speedup vs baseline: 1.0947x; 1.0947x over previous
import jax
import jax.numpy as jnp
from jax import lax
from jax.experimental import pallas as pl
from jax.experimental.pallas import tpu as pltpu
from jax.experimental.pallas import tpu_sc as plsc

BLOCK = 8192
EMBED = 1024

_info = plsc.get_sparse_core_info()
_NC, _NS = _info.num_cores, _info.num_subcores
_NW = _NC * _NS
_ROWS_PER_W = BLOCK // _NW
_CHUNK = 8
_NSTEPS = _ROWS_PER_W // _CHUNK
_NBUF = 12
_AHEAD = 8


def _copy_body(pe_hbm, out_hbm, *rest):
    bufs = rest[:_NBUF]
    sem_in, sem_out = rest[_NBUF], rest[_NBUF + 1]
    wid = lax.axis_index("s") * _NC + lax.axis_index("c")
    base = wid * _ROWS_PER_W

    def start_in(i):
        return pltpu.async_copy(
            pe_hbm.at[pl.ds(base + i * _CHUNK, _CHUNK)], bufs[i % _NBUF], sem_in)

    def start_out(i):
        return pltpu.async_copy(
            bufs[i % _NBUF], out_hbm.at[pl.ds(base + i * _CHUNK, _CHUNK)], sem_out)

    copies_in = [None] * _NSTEPS
    copies_out = [None] * _NSTEPS
    out_waited = [False] * _NSTEPS
    for i in range(_AHEAD):
        copies_in[i] = start_in(i)
    for i in range(_NSTEPS):
        copies_in[i].wait()
        copies_out[i] = start_out(i)
        j = i + _AHEAD
        if j < _NSTEPS:
            prev = j - _NBUF
            if prev >= 0:
                copies_out[prev].wait()
                out_waited[prev] = True
            copies_in[j] = start_in(j)
    for i in range(_NSTEPS):
        if not out_waited[i]:
            copies_out[i].wait()


def _sc_copy(pe):
    mesh = plsc.VectorSubcoreMesh(core_axis_name="c", subcore_axis_name="s")
    return pl.kernel(
        _copy_body,
        mesh=mesh,
        out_type=jax.ShapeDtypeStruct((BLOCK, EMBED), jnp.float32),
        scratch_types=(
            [pltpu.VMEM((_CHUNK, EMBED), jnp.float32) for _ in range(_NBUF)]
            + [pltpu.SemaphoreType.DMA, pltpu.SemaphoreType.DMA]
        ),
    )(pe)


def kernel(x, pe):
    return _sc_copy(pe)


# speedup vs baseline: 1.1352x; 1.0370x over previous
import jax
import jax.numpy as jnp
from jax import lax
from jax.experimental import pallas as pl
from jax.experimental.pallas import tpu as pltpu
from jax.experimental.pallas import tpu_sc as plsc

BLOCK = 8192
EMBED = 1024

_info = plsc.get_sparse_core_info()
_NC, _NS = _info.num_cores, _info.num_subcores
_NW = _NC * _NS
_ROWS_PER_W = BLOCK // _NW
_CHUNK = 16
_NSTEPS = _ROWS_PER_W // _CHUNK
_NBUF = 7
_AHEAD = 5


def _copy_body(pe_hbm, out_hbm, *rest):
    bufs = rest[:_NBUF]
    sem_in, sem_out = rest[_NBUF], rest[_NBUF + 1]
    wid = lax.axis_index("s") * _NC + lax.axis_index("c")
    base = wid * _ROWS_PER_W

    def start_in(i):
        return pltpu.async_copy(
            pe_hbm.at[pl.ds(base + i * _CHUNK, _CHUNK)], bufs[i % _NBUF], sem_in)

    def start_out(i):
        return pltpu.async_copy(
            bufs[i % _NBUF], out_hbm.at[pl.ds(base + i * _CHUNK, _CHUNK)], sem_out)

    copies_in = [None] * _NSTEPS
    copies_out = [None] * _NSTEPS
    out_waited = [False] * _NSTEPS
    for i in range(_AHEAD):
        copies_in[i] = start_in(i)
    for i in range(_NSTEPS):
        copies_in[i].wait()
        copies_out[i] = start_out(i)
        j = i + _AHEAD
        if j < _NSTEPS:
            prev = j - _NBUF
            if prev >= 0:
                copies_out[prev].wait()
                out_waited[prev] = True
            copies_in[j] = start_in(j)
    for i in range(_NSTEPS):
        if not out_waited[i]:
            copies_out[i].wait()


def _sc_copy(pe):
    mesh = plsc.VectorSubcoreMesh(core_axis_name="c", subcore_axis_name="s")
    return pl.kernel(
        _copy_body,
        mesh=mesh,
        out_type=jax.ShapeDtypeStruct((BLOCK, EMBED), jnp.float32),
        scratch_types=(
            [pltpu.VMEM((_CHUNK, EMBED), jnp.float32) for _ in range(_NBUF)]
            + [pltpu.SemaphoreType.DMA, pltpu.SemaphoreType.DMA]
        ),
    )(pe)


def kernel(x, pe):
    return _sc_copy(pe)


# speedup vs baseline: 1.1662x; 1.0274x over previous
import jax
import jax.numpy as jnp
from jax import lax
from jax._src.pallas import mpmd
from jax.experimental import pallas as pl
from jax.experimental.pallas import tpu as pltpu
from jax.experimental.pallas import tpu_sc as plsc

BLOCK = 8192
EMBED = 1024

_info = plsc.get_sparse_core_info()
_NC, _NS = _info.num_cores, _info.num_subcores
_NW = _NC * _NS

_SCS_ROWS = 512
_TEC_ROWS = BLOCK - _NC * _SCS_ROWS
_ROWS_PER_W = _TEC_ROWS // _NW
_CHUNK = 16
_NSTEPS = _ROWS_PER_W // _CHUNK
_NBUF = 7
_AHEAD = 5

_SCS_CHUNK = 64
_SCS_STEPS = _SCS_ROWS // _SCS_CHUNK


def _tec_fn(pe_hbm, out_hbm, spmem):
    del spmem

    def body(*rest):
        bufs = rest[:_NBUF]
        sem_in, sem_out = rest[_NBUF], rest[_NBUF + 1]
        wid = lax.axis_index("s") * _NC + lax.axis_index("c")
        base = wid * _ROWS_PER_W

        def start_in(i):
            return pltpu.async_copy(
                pe_hbm.at[pl.ds(base + i * _CHUNK, _CHUNK)],
                bufs[i % _NBUF], sem_in)

        def start_out(i):
            return pltpu.async_copy(
                bufs[i % _NBUF],
                out_hbm.at[pl.ds(base + i * _CHUNK, _CHUNK)], sem_out)

        copies_in = [None] * _NSTEPS
        copies_out = [None] * _NSTEPS
        out_waited = [False] * _NSTEPS
        for i in range(_AHEAD):
            copies_in[i] = start_in(i)
        for i in range(_NSTEPS):
            copies_in[i].wait()
            copies_out[i] = start_out(i)
            j = i + _AHEAD
            if j < _NSTEPS:
                prev = j - _NBUF
                if prev >= 0:
                    copies_out[prev].wait()
                    out_waited[prev] = True
                copies_in[j] = start_in(j)
        for i in range(_NSTEPS):
            if not out_waited[i]:
                copies_out[i].wait()

    pl.run_scoped(
        body,
        *([pltpu.VMEM((_CHUNK, EMBED), jnp.float32)] * _NBUF),
        pltpu.SemaphoreType.DMA,
        pltpu.SemaphoreType.DMA,
    )


def _scs_fn(pe_hbm, out_hbm, spmem):
    def body(sem_in, sem_out):
        c = lax.axis_index("c")
        base = _TEC_ROWS + c * _SCS_ROWS

        def start_in(i):
            return pltpu.async_copy(
                pe_hbm.at[pl.ds(base + i * _SCS_CHUNK, _SCS_CHUNK)],
                spmem.at[i % 2], sem_in)

        def start_out(i):
            return pltpu.async_copy(
                spmem.at[i % 2],
                out_hbm.at[pl.ds(base + i * _SCS_CHUNK, _SCS_CHUNK)], sem_out)

        copies_in = [None] * _SCS_STEPS
        copies_out = [None] * _SCS_STEPS
        copies_in[0] = start_in(0)
        for i in range(_SCS_STEPS):
            if i > 0:
                copies_out[i - 1].wait()
            copies_in[i].wait()
            copies_out[i] = start_out(i)
            if i + 1 < _SCS_STEPS:
                copies_in[i + 1] = start_in(i + 1)
        copies_out[_SCS_STEPS - 1].wait()

    pl.run_scoped(body, pltpu.SemaphoreType.DMA, pltpu.SemaphoreType.DMA)


def _sc_copy(pe):
    vec_mesh = plsc.VectorSubcoreMesh(core_axis_name="c", subcore_axis_name="s")
    scs_mesh = plsc.ScalarSubcoreMesh(axis_name="c")
    return mpmd.mpmd_map(
        [(scs_mesh, _scs_fn), (vec_mesh, _tec_fn)],
        out_types=[jax.ShapeDtypeStruct((BLOCK, EMBED), jnp.float32)],
        scratch_types=[
            pltpu.VMEM_SHARED((2, _SCS_CHUNK, EMBED), jnp.float32),
        ],
    )(pe)[0]


def kernel(x, pe):
    return _sc_copy(pe)
